# baseline (speedup 1.0000x reference)
import functools

import jax
import jax.numpy as jnp
from jax import lax
from jax.experimental import pallas as pl
from jax.experimental.pallas import tpu as pltpu
from jax.experimental.pallas import tpu_sc as plsc

VOCAB = 1000
BATCH, TIME = 1024, 20
NUM_CORES = 2
NUM_SUBCORES = 16
NW = NUM_CORES * NUM_SUBCORES
B_PER_W = BATCH // NW
NBUF = 4

_MESH = plsc.VectorSubcoreMesh(core_axis_name="c", subcore_axis_name="s")


@functools.partial(
    pl.kernel,
    mesh=_MESH,
    out_type=jax.ShapeDtypeStruct((BATCH, TIME, VOCAB), jnp.float32),
    scratch_types=[
        pltpu.VMEM((B_PER_W, TIME), jnp.int32),
        pltpu.VMEM((NBUF, TIME, VOCAB), jnp.float32),
    ]
    + [pltpu.SemaphoreType.DMA] * NBUF
    + [pltpu.SemaphoreType.DMA] * NBUF,
    compiler_params=pltpu.CompilerParams(use_tc_tiling_on_sc=False),
)
def _gather_rows(idx_hbm, table_hbm, out_hbm, idx_v, bufs, *sems):
    gsems = sems[:NBUF]
    wsems = sems[NBUF:]
    wid = lax.axis_index("s") * NUM_CORES + lax.axis_index("c")
    base = wid * B_PER_W
    pltpu.sync_copy(idx_hbm.at[wid], idx_v)
    gh = [None] * B_PER_W
    wh = [None] * B_PER_W
    for c in range(min(NBUF, B_PER_W)):
        gh[c] = pltpu.async_copy(
            table_hbm.at[idx_v.at[c]], bufs.at[c], gsems[c]
        )
    for c in range(B_PER_W):
        b = c % NBUF
        gh[c].wait()
        wh[c] = pltpu.async_copy(bufs.at[b], out_hbm.at[base + c], wsems[b])
        nxt = c + NBUF
        if nxt < B_PER_W:
            wh[c].wait()
            gh[nxt] = pltpu.async_copy(
                table_hbm.at[idx_v.at[nxt]], bufs.at[b], gsems[b]
            )
    for c in range(B_PER_W - NBUF, B_PER_W):
        if wh[c] is not None:
            wh[c].wait()


def kernel(idx, table):
    idx3 = idx.reshape(NW, B_PER_W, TIME).astype(jnp.int32)
    return _gather_rows(idx3, table)
